# baseline (speedup 1.0000x reference)
import functools

import jax
import jax.numpy as jnp
import numpy as np
from jax import lax
from jax.experimental import pallas as pl
from jax.experimental.pallas import tpu as pltpu
from jax.experimental.pallas import tpu_sc as plsc

N = 50000
H = 512
W = 512
R = 6
K = 2 * R + 1

G = 512
NP = 50176
NB = NP // G

PCH = 12544
LCAP = 296
CAP = 16 * LCAP
CAPR = CAP // 128
BANDH = 16
FBN = BANDH * W
GUARD = 8
FBA = FBN + 32


def _prep_body(xyz_ref, scl_ref, rot_ref, feat_ref, opac_ref,
               px_ref, py_ref, a_ref, b_ref, c_ref,
               fr_ref, fg_ref, fb_ref):
    xy = jnp.tanh(xyz_ref[...])
    px_ref[...] = (0.5 * W) * (xy[:, 0] + 1.0)
    py_ref[...] = (0.5 * H) * (xy[:, 1] + 1.0)
    scl = jnp.abs(scl_ref[...] + 0.5)
    s1 = scl[:, 0]
    s2 = scl[:, 1]
    rot = jax.nn.sigmoid(rot_ref[:, 0]) * (2.0 * np.pi)
    c = jnp.cos(rot)
    s = jnp.sin(rot)
    S00 = s1 * s1 * c * c + s2 * s2 * s * s
    S01 = (s1 * s1 - s2 * s2) * s * c
    S11 = s1 * s1 * s * s + s2 * s2 * c * c
    det = jnp.maximum(S00 * S11 - S01 * S01, 1e-12)
    a_ref[...] = S11 / det
    b_ref[...] = -S01 / det
    c_ref[...] = S00 / det
    f = feat_ref[...] * opac_ref[...]
    fr_ref[...] = f[:, 0]
    fg_ref[...] = f[:, 1]
    fb_ref[...] = f[:, 2]


def _prep_call(xyz, scl, rot, feat, opac):
    outs = [jax.ShapeDtypeStruct((NP,), jnp.float32)] * 8
    in_spec = lambda bs: pl.BlockSpec(bs, lambda i: (i, 0))
    out_spec = pl.BlockSpec((G,), lambda i: (i,))
    return pl.pallas_call(
        _prep_body,
        grid=(NB,),
        in_specs=[in_spec((G, 2)), in_spec((G, 2)), in_spec((G, 1)),
                  in_spec((G, 3)), in_spec((G, 1))],
        out_specs=[out_spec] * 8,
        out_shape=outs,
    )(xyz, scl, rot, feat, opac)


def _band_body(px_hbm, py_hbm, a_hbm, b_hbm, c_hbm, fr_hbm, fg_hbm, fb_hbm,
               out_hbm,
               pybuf, idx2, tmpb, pk,
               im_r, im_g, im_b):
    cidx = lax.axis_index("c")
    sidx = lax.axis_index("s")
    wid = cidx * 16 + sidx
    r0 = wid * BANDH
    i16 = lax.iota(jnp.int32, 16)
    z16f = jnp.zeros((16,), jnp.float32)
    z16i = jnp.zeros((16,), jnp.int32)

    def zero(i, carry):
        im_r[pl.ds(i * 16, 16)] = z16f
        im_g[pl.ds(i * 16, 16)] = z16f
        im_b[pl.ds(i * 16, 16)] = z16f
        return carry

    lax.fori_loop(0, FBA // 16, zero, 0)

    def prefill(i, carry):
        idx2[pl.ds(i * 16, 16)] = z16i
        return carry

    lax.fori_loop(0, CAP // 16, prefill, 0)

    one16 = jnp.full((16,), 1, jnp.int32)
    lane_base = i16 * LCAP

    def chunk_scan(ci, clv0):
        pltpu.sync_copy(py_hbm.at[pl.ds(ci * PCH, PCH)], pybuf)

        def grp(i, clv):
            pyv = pybuf[pl.ds(i * 16, 16)]
            cyv = pyv.astype(jnp.int32)
            m = (cyv >= r0 - R) & (cyv <= r0 + BANDH - 1 + R)
            g = ci * PCH + i * 16 + i16
            pos = jnp.where(m, lane_base + clv, jnp.full((16,), CAP,
                                                         jnp.int32) + i16)
            plsc.store_scatter(idx2, [pos], g)
            return clv + jnp.where(m, one16, z16i)

        return lax.fori_loop(0, PCH // 16, grp, clv0)

    clv = lax.fori_loop(0, NP // PCH, chunk_scan, z16i)

    for j, src in enumerate((px_hbm, py_hbm, a_hbm, b_hbm, c_hbm,
                             fr_hbm, fg_hbm, fb_hbm)):
        def gath(r, carry, src=src):
            irow = idx2.at[pl.ds(r * 128, 128)]
            pltpu.sync_copy(src.at[irow], tmpb.at[pl.ds(r * 128, 128)])
            return carry

        lax.fori_loop(0, CAPR, gath, 0)

        def ilv(i, carry, j=j):
            v = tmpb[pl.ds(i * 16, 16)]
            plsc.store_scatter(pk, [(i * 16 + i16) * 8 + j], v)
            return carry

        lax.fori_loop(0, CAP // 16, ilv, 0)

    lane13 = i16 <= (K - 1)

    for l in range(16):
        cnt_l = clv[l]

        def one_g(q, c2, l=l):
            pv = pk[pl.ds((l * LCAP + q) * 8, 16)]
            px1 = pv[0]
            py1 = pv[1]
            a1 = pv[2]
            b1 = pv[3]
            c1 = pv[4]
            fr1 = pv[5]
            fg1 = pv[6]
            fb1 = pv[7]
            cx1 = px1.astype(jnp.int32)
            cy1 = py1.astype(jnp.int32)
            xv = cx1 - R + i16
            dx = xv.astype(jnp.float32) + 0.5 - px1
            sigA = (0.5 * a1) * dx * dx
            bdx = b1 * dx
            halfC = 0.5 * c1
            xin = lane13 & (xv >= 0) & (xv < W)
            ibase0 = cx1 - R + GUARD + i16
            rlo = jnp.maximum(cy1 - R, r0)
            rhi = jnp.minimum(cy1 + R, r0 + BANDH - 1)

            def row(ry, c3):
                dy = ry.astype(jnp.float32) + 0.5 - py1
                sig = sigA + bdx * dy + (halfC * dy) * dy
                al = jnp.exp(-sig)
                m = xin & (sig >= 0.0)
                al = jnp.where(m, al, z16f)
                idx = (ry - r0) * W + ibase0
                plsc.addupdate_scatter(im_r, [idx], al * fr1)
                plsc.addupdate_scatter(im_g, [idx], al * fg1)
                plsc.addupdate_scatter(im_b, [idx], al * fb1)
                return c3

            lax.fori_loop(rlo, rhi + 1, row, c2)
            return c2

        lax.fori_loop(0, cnt_l, one_g, 0)

    def clipv(i, carry):
        for im in (im_r, im_g, im_b):
            im[pl.ds(i * 16, 16)] = jnp.clip(im[pl.ds(i * 16, 16)], 0.0, 1.0)
        return carry

    lax.fori_loop(0, FBA // 16, clipv, 0)
    for ch, im in enumerate((im_r, im_g, im_b)):
        def orow(yl, carry, im=im, ch=ch):
            pltpu.sync_copy(im.at[pl.ds(yl * W + GUARD, W)],
                            out_hbm.at[ch, r0 + yl])
            return carry

        lax.fori_loop(0, BANDH, orow, 0)


def _band_scatter(px, py, a, b, c, fr, fg, fb):
    mesh = plsc.VectorSubcoreMesh(core_axis_name="c", subcore_axis_name="s")
    fn = pl.kernel(
        _band_body,
        out_type=jax.ShapeDtypeStruct((3, H, W), jnp.float32),
        mesh=mesh,
        scratch_types=[
            pltpu.VMEM((PCH,), jnp.float32),
            pltpu.VMEM((CAP + 16,), jnp.int32),
            pltpu.VMEM((CAP,), jnp.float32),
            pltpu.VMEM((CAP * 8 + 16,), jnp.float32),
        ] + [
            pltpu.VMEM((FBA,), jnp.float32),
            pltpu.VMEM((FBA,), jnp.float32),
            pltpu.VMEM((FBA,), jnp.float32),
        ],
        compiler_params=pltpu.CompilerParams(
            use_tc_tiling_on_sc=False, needs_layout_passes=False),
    )
    return fn(px, py, a, b, c, fr, fg, fb)


def _pad_inputs(xyz, scl, rot, feat, opac):
    p = NP - N
    t = (jnp.arange(p, dtype=jnp.float32) + 0.5) / p
    tx = jnp.arctanh(t * 1.98 - 0.99)
    ty = jnp.arctanh(((t * 37.0) % 1.0) * 1.98 - 0.99)
    xyz_p = jnp.concatenate([xyz, jnp.stack([tx, ty], axis=-1)], axis=0)
    scl_p = jnp.concatenate([scl, jnp.zeros((p, 2), jnp.float32)], axis=0)
    rot_p = jnp.concatenate([rot, jnp.zeros((p, 1), jnp.float32)], axis=0)
    feat_p = jnp.concatenate([feat, jnp.zeros((p, 3), jnp.float32)], axis=0)
    opac_p = jnp.concatenate([opac, jnp.zeros((p, 1), jnp.float32)], axis=0)
    return xyz_p, scl_p, rot_p, feat_p, opac_p


def kernel(_xyz, _scaling, _rotation, _features_dc, _opacity):
    xyz, scl, rot, feat, opac = _pad_inputs(
        _xyz, _scaling, _rotation, _features_dc, _opacity)
    px, py, a, b, c, fr, fg, fb = _prep_call(xyz, scl, rot, feat, opac)
    img = _band_scatter(px, py, a, b, c, fr, fg, fb)
    return img.reshape(1, 3, H, W)


# speedup vs baseline: 2.2663x; 2.2663x over previous
import functools

import jax
import jax.numpy as jnp
import numpy as np
from jax import lax
from jax.experimental import pallas as pl
from jax.experimental.pallas import tpu as pltpu
from jax.experimental.pallas import tpu_sc as plsc

N = 50000
H = 512
W = 512
R = 6
K = 2 * R + 1

G = 512
NP = 50176
NB = NP // G

PCH = 12544
LCAP = 296
CAP = 16 * LCAP
CAPR = CAP // 128
BANDH = 16
FBN = BANDH * W
GUARD = 8
FBA = FBN + 32


def _prep_body(xyz_ref, scl_ref, rot_ref, feat_ref, opac_ref,
               px_ref, py_ref, a_ref, b_ref, c_ref,
               fr_ref, fg_ref, fb_ref):
    xy = jnp.tanh(xyz_ref[...])
    px_ref[...] = (0.5 * W) * (xy[:, 0] + 1.0)
    py_ref[...] = (0.5 * H) * (xy[:, 1] + 1.0)
    scl = jnp.abs(scl_ref[...] + 0.5)
    s1 = scl[:, 0]
    s2 = scl[:, 1]
    rot = jax.nn.sigmoid(rot_ref[:, 0]) * (2.0 * np.pi)
    c = jnp.cos(rot)
    s = jnp.sin(rot)
    S00 = s1 * s1 * c * c + s2 * s2 * s * s
    S01 = (s1 * s1 - s2 * s2) * s * c
    S11 = s1 * s1 * s * s + s2 * s2 * c * c
    det = jnp.maximum(S00 * S11 - S01 * S01, 1e-12)
    a_ref[...] = S11 / det
    b_ref[...] = -S01 / det
    c_ref[...] = S00 / det
    f = feat_ref[...] * opac_ref[...]
    fr_ref[...] = f[:, 0]
    fg_ref[...] = f[:, 1]
    fb_ref[...] = f[:, 2]


def _prep_call(xyz, scl, rot, feat, opac):
    outs = [jax.ShapeDtypeStruct((NP,), jnp.float32)] * 8
    in_spec = lambda bs: pl.BlockSpec(bs, lambda i: (i, 0))
    out_spec = pl.BlockSpec((G,), lambda i: (i,))
    return pl.pallas_call(
        _prep_body,
        grid=(NB,),
        in_specs=[in_spec((G, 2)), in_spec((G, 2)), in_spec((G, 1)),
                  in_spec((G, 3)), in_spec((G, 1))],
        out_specs=[out_spec] * 8,
        out_shape=outs,
    )(xyz, scl, rot, feat, opac)


def _band_body(px_hbm, py_hbm, a_hbm, b_hbm, c_hbm, fr_hbm, fg_hbm, fb_hbm,
               out_hbm,
               pybuf, idx2, pxb, pyb, ab, bb, cb, frb, fgb, fbb,
               im_r, im_g, im_b):
    cidx = lax.axis_index("c")
    sidx = lax.axis_index("s")
    wid = cidx * 16 + sidx
    r0 = wid * BANDH
    i16 = lax.iota(jnp.int32, 16)
    z16f = jnp.zeros((16,), jnp.float32)
    z16i = jnp.zeros((16,), jnp.int32)

    def zero(i, carry):
        im_r[pl.ds(i * 16, 16)] = z16f
        im_g[pl.ds(i * 16, 16)] = z16f
        im_b[pl.ds(i * 16, 16)] = z16f
        return carry

    lax.fori_loop(0, FBA // 16, zero, 0)

    def prefill(i, carry):
        idx2[pl.ds(i * 16, 16)] = z16i
        return carry

    lax.fori_loop(0, CAP // 16, prefill, 0)

    one16 = jnp.full((16,), 1, jnp.int32)
    lane_base = i16 * LCAP

    def chunk_scan(ci, clv0):
        pltpu.sync_copy(py_hbm.at[pl.ds(ci * PCH, PCH)], pybuf)

        def grp(i, clv):
            pyv = pybuf[pl.ds(i * 16, 16)]
            cyv = pyv.astype(jnp.int32)
            m = (cyv >= r0 - R) & (cyv <= r0 + BANDH - 1 + R)
            g = ci * PCH + i * 16 + i16
            pos = jnp.where(m, lane_base + clv, jnp.full((16,), CAP,
                                                         jnp.int32) + i16)
            plsc.store_scatter(idx2, [pos], g)
            return clv + jnp.where(m, one16, z16i)

        return lax.fori_loop(0, PCH // 16, grp, clv0)

    clv = lax.fori_loop(0, NP // PCH, chunk_scan, z16i)

    def gath(r, carry):
        irow = idx2.at[pl.ds(r * 128, 128)]
        dst = pl.ds(r * 128, 128)
        pltpu.sync_copy(px_hbm.at[irow], pxb.at[dst])
        pltpu.sync_copy(py_hbm.at[irow], pyb.at[dst])
        pltpu.sync_copy(a_hbm.at[irow], ab.at[dst])
        pltpu.sync_copy(b_hbm.at[irow], bb.at[dst])
        pltpu.sync_copy(c_hbm.at[irow], cb.at[dst])
        pltpu.sync_copy(fr_hbm.at[irow], frb.at[dst])
        pltpu.sync_copy(fg_hbm.at[irow], fgb.at[dst])
        pltpu.sync_copy(fb_hbm.at[irow], fbb.at[dst])
        return carry

    lax.fori_loop(0, CAPR, gath, 0)

    lane13 = i16 <= (K - 1)

    for l in range(16):
        cnt_l = clv[l]

        def one_g(q, c2, l=l):
            slot = l * LCAP + q
            px1 = pxb[pl.ds(slot, 16)][0]
            py1 = pyb[pl.ds(slot, 16)][0]
            a1 = ab[pl.ds(slot, 16)][0]
            b1 = bb[pl.ds(slot, 16)][0]
            c1 = cb[pl.ds(slot, 16)][0]
            fr1 = frb[pl.ds(slot, 16)][0]
            fg1 = fgb[pl.ds(slot, 16)][0]
            fb1 = fbb[pl.ds(slot, 16)][0]
            cx1 = px1.astype(jnp.int32)
            cy1 = py1.astype(jnp.int32)
            xv = cx1 - R + i16
            dx = xv.astype(jnp.float32) + 0.5 - px1
            sigA = (0.5 * a1) * dx * dx
            bdx = b1 * dx
            halfC = 0.5 * c1
            xin = lane13 & (xv >= 0) & (xv < W)
            ibase0 = cx1 - R + GUARD + i16
            rlo = jnp.maximum(cy1 - R, r0)
            rhi = jnp.minimum(cy1 + R, r0 + BANDH - 1)

            def row(ry, c3):
                dy = ry.astype(jnp.float32) + 0.5 - py1
                sig = sigA + bdx * dy + (halfC * dy) * dy
                al = jnp.exp(-sig)
                m = xin & (sig >= 0.0)
                al = jnp.where(m, al, z16f)
                idx = (ry - r0) * W + ibase0
                plsc.addupdate_scatter(im_r, [idx], al * fr1)
                plsc.addupdate_scatter(im_g, [idx], al * fg1)
                plsc.addupdate_scatter(im_b, [idx], al * fb1)
                return c3

            lax.fori_loop(rlo, rhi + 1, row, c2)
            return c2

        lax.fori_loop(0, cnt_l, one_g, 0)

    def clipv(i, carry):
        for im in (im_r, im_g, im_b):
            im[pl.ds(i * 16, 16)] = jnp.clip(im[pl.ds(i * 16, 16)], 0.0, 1.0)
        return carry

    lax.fori_loop(0, FBA // 16, clipv, 0)
    for ch, im in enumerate((im_r, im_g, im_b)):
        def orow(yl, carry, im=im, ch=ch):
            pltpu.sync_copy(im.at[pl.ds(yl * W + GUARD, W)],
                            out_hbm.at[ch, r0 + yl])
            return carry

        lax.fori_loop(0, BANDH, orow, 0)


def _band_scatter(px, py, a, b, c, fr, fg, fb):
    mesh = plsc.VectorSubcoreMesh(core_axis_name="c", subcore_axis_name="s")
    fn = pl.kernel(
        _band_body,
        out_type=jax.ShapeDtypeStruct((3, H, W), jnp.float32),
        mesh=mesh,
        scratch_types=[
            pltpu.VMEM((PCH,), jnp.float32),
            pltpu.VMEM((CAP + 16,), jnp.int32),
        ] + [pltpu.VMEM((CAP + 16,), jnp.float32)] * 8 + [
            pltpu.VMEM((FBA,), jnp.float32),
            pltpu.VMEM((FBA,), jnp.float32),
            pltpu.VMEM((FBA,), jnp.float32),
        ],
        compiler_params=pltpu.CompilerParams(
            use_tc_tiling_on_sc=False, needs_layout_passes=False),
    )
    return fn(px, py, a, b, c, fr, fg, fb)


def _pad_inputs(xyz, scl, rot, feat, opac):
    p = NP - N
    t = (jnp.arange(p, dtype=jnp.float32) + 0.5) / p
    tx = jnp.arctanh(t * 1.98 - 0.99)
    ty = jnp.arctanh(((t * 37.0) % 1.0) * 1.98 - 0.99)
    xyz_p = jnp.concatenate([xyz, jnp.stack([tx, ty], axis=-1)], axis=0)
    scl_p = jnp.concatenate([scl, jnp.zeros((p, 2), jnp.float32)], axis=0)
    rot_p = jnp.concatenate([rot, jnp.zeros((p, 1), jnp.float32)], axis=0)
    feat_p = jnp.concatenate([feat, jnp.zeros((p, 3), jnp.float32)], axis=0)
    opac_p = jnp.concatenate([opac, jnp.zeros((p, 1), jnp.float32)], axis=0)
    return xyz_p, scl_p, rot_p, feat_p, opac_p


def kernel(_xyz, _scaling, _rotation, _features_dc, _opacity):
    xyz, scl, rot, feat, opac = _pad_inputs(
        _xyz, _scaling, _rotation, _features_dc, _opacity)
    px, py, a, b, c, fr, fg, fb = _prep_call(xyz, scl, rot, feat, opac)
    img = _band_scatter(px, py, a, b, c, fr, fg, fb)
    return img.reshape(1, 3, H, W)


# speedup vs baseline: 2.2666x; 1.0001x over previous
import jax
import jax.numpy as jnp
import numpy as np
from jax import lax
from jax.experimental import pallas as pl
from jax.experimental.pallas import tpu as pltpu
from jax.experimental.pallas import tpu_sc as plsc

N = 50000
H = 512
W = 512
R = 6
K = 2 * R + 1

G = 512
NP = 50176
NB = NP // G

PCH = 12544
LCAP = 296
CAP = 16 * LCAP
CAPR = CAP // 128
BANDH = 16
FBN = BANDH * W
GUARD = 8
FBA = FBN + 32


def _prep_body(xyz_ref, scl_ref, rot_ref, feat_ref, opac_ref,
               px_ref, py_ref, a_ref, b_ref, c_ref,
               fr_ref, fg_ref, fb_ref):
    xy = jnp.tanh(xyz_ref[...])
    px_ref[...] = (0.5 * W) * (xy[:, 0] + 1.0)
    py_ref[...] = (0.5 * H) * (xy[:, 1] + 1.0)
    scl = jnp.abs(scl_ref[...] + 0.5)
    s1 = scl[:, 0]
    s2 = scl[:, 1]
    rot = jax.nn.sigmoid(rot_ref[:, 0]) * (2.0 * np.pi)
    c = jnp.cos(rot)
    s = jnp.sin(rot)
    S00 = s1 * s1 * c * c + s2 * s2 * s * s
    S01 = (s1 * s1 - s2 * s2) * s * c
    S11 = s1 * s1 * s * s + s2 * s2 * c * c
    det = jnp.maximum(S00 * S11 - S01 * S01, 1e-12)
    a_ref[...] = S11 / det
    b_ref[...] = -S01 / det
    c_ref[...] = S00 / det
    f = feat_ref[...] * opac_ref[...]
    fr_ref[...] = f[:, 0]
    fg_ref[...] = f[:, 1]
    fb_ref[...] = f[:, 2]


def _prep_call(xyz, scl, rot, feat, opac):
    outs = [jax.ShapeDtypeStruct((NP,), jnp.float32)] * 8
    in_spec = lambda bs: pl.BlockSpec(bs, lambda i: (i, 0))
    out_spec = pl.BlockSpec((G,), lambda i: (i,))
    return pl.pallas_call(
        _prep_body,
        grid=(NB,),
        in_specs=[in_spec((G, 2)), in_spec((G, 2)), in_spec((G, 1)),
                  in_spec((G, 3)), in_spec((G, 1))],
        out_specs=[out_spec] * 8,
        out_shape=outs,
    )(xyz, scl, rot, feat, opac)


def _band_body(px_hbm, py_hbm, a_hbm, b_hbm, c_hbm, fr_hbm, fg_hbm, fb_hbm,
               out_hbm,
               pybuf, idx2, pxb, pyb, ab, bb, cb, frb, fgb, fbb,
               im_r, im_g, im_b):
    cidx = lax.axis_index("c")
    sidx = lax.axis_index("s")
    wid = cidx * 16 + sidx
    r0 = wid * BANDH
    i16 = lax.iota(jnp.int32, 16)
    z16f = jnp.zeros((16,), jnp.float32)
    z16i = jnp.zeros((16,), jnp.int32)

    def zero(i, carry):
        im_r[pl.ds(i * 16, 16)] = z16f
        im_g[pl.ds(i * 16, 16)] = z16f
        im_b[pl.ds(i * 16, 16)] = z16f
        return carry

    lax.fori_loop(0, FBA // 16, zero, 0)

    def prefill(i, carry):
        idx2[pl.ds(i * 16, 16)] = z16i
        return carry

    lax.fori_loop(0, CAP // 16, prefill, 0)

    one16 = jnp.full((16,), 1, jnp.int32)
    lane_base = i16 * LCAP

    def chunk_scan(ci, clv0):
        pltpu.sync_copy(py_hbm.at[pl.ds(ci * PCH, PCH)], pybuf)

        def grp(i, clv):
            pyv = pybuf[pl.ds(i * 16, 16)]
            cyv = pyv.astype(jnp.int32)
            m = (cyv >= r0 - R) & (cyv <= r0 + BANDH - 1 + R)
            g = ci * PCH + i * 16 + i16
            pos = jnp.where(m, lane_base + clv, jnp.full((16,), CAP,
                                                         jnp.int32) + i16)
            plsc.store_scatter(idx2, [pos], g)
            return clv + jnp.where(m, one16, z16i)

        return lax.fori_loop(0, PCH // 16, grp, clv0)

    clv = lax.fori_loop(0, NP // PCH, chunk_scan, z16i)

    def gath(r, carry):
        irow = idx2.at[pl.ds(r * 128, 128)]
        dst = pl.ds(r * 128, 128)
        pltpu.sync_copy(px_hbm.at[irow], pxb.at[dst])
        pltpu.sync_copy(py_hbm.at[irow], pyb.at[dst])
        pltpu.sync_copy(a_hbm.at[irow], ab.at[dst])
        pltpu.sync_copy(b_hbm.at[irow], bb.at[dst])
        pltpu.sync_copy(c_hbm.at[irow], cb.at[dst])
        pltpu.sync_copy(fr_hbm.at[irow], frb.at[dst])
        pltpu.sync_copy(fg_hbm.at[irow], fgb.at[dst])
        pltpu.sync_copy(fb_hbm.at[irow], fbb.at[dst])
        return carry

    lax.fori_loop(0, CAPR, gath, 0)

    lane13 = i16 <= (K - 1)

    for l in range(16):
        cnt_l = clv[l]

        def one_g(q, c2, l=l):
            slot = l * LCAP + q
            px1 = pxb[pl.ds(slot, 16)][0]
            py1 = pyb[pl.ds(slot, 16)][0]
            a1 = ab[pl.ds(slot, 16)][0]
            b1 = bb[pl.ds(slot, 16)][0]
            c1 = cb[pl.ds(slot, 16)][0]
            fr1 = frb[pl.ds(slot, 16)][0]
            fg1 = fgb[pl.ds(slot, 16)][0]
            fb1 = fbb[pl.ds(slot, 16)][0]
            cx1 = px1.astype(jnp.int32)
            cy1 = py1.astype(jnp.int32)
            xv = cx1 - R + i16
            dx = xv.astype(jnp.float32) + 0.5 - px1
            sigA = (0.5 * a1) * dx * dx
            bdx = b1 * dx
            halfC = 0.5 * c1
            xin = lane13 & (xv >= 0) & (xv < W)
            ibase0 = cx1 - R + GUARD + i16
            rlo = jnp.maximum(cy1 - R, r0)
            rhi = jnp.minimum(cy1 + R, r0 + BANDH - 1)

            def row(ry, c3):
                dy = ry.astype(jnp.float32) + 0.5 - py1
                sig = sigA + bdx * dy + (halfC * dy) * dy
                al = jnp.exp(-sig)
                m = xin & (sig >= 0.0)
                al = jnp.where(m, al, z16f)
                idx = (ry - r0) * W + ibase0
                plsc.addupdate_scatter(im_r, [idx], al * fr1)
                plsc.addupdate_scatter(im_g, [idx], al * fg1)
                plsc.addupdate_scatter(im_b, [idx], al * fb1)
                return c3

            lax.fori_loop(rlo, rhi + 1, row, c2)
            return c2

        lax.fori_loop(0, cnt_l, one_g, 0)

    def clipv(i, carry):
        for im in (im_r, im_g, im_b):
            im[pl.ds(i * 16, 16)] = jnp.clip(im[pl.ds(i * 16, 16)], 0.0, 1.0)
        return carry

    lax.fori_loop(0, FBA // 16, clipv, 0)
    for ch, im in enumerate((im_r, im_g, im_b)):
        def orow(yl, carry, im=im, ch=ch):
            pltpu.sync_copy(im.at[pl.ds(yl * W + GUARD, W)],
                            out_hbm.at[ch, r0 + yl])
            return carry

        lax.fori_loop(0, BANDH, orow, 0)


def _band_scatter(px, py, a, b, c, fr, fg, fb):
    mesh = plsc.VectorSubcoreMesh(core_axis_name="c", subcore_axis_name="s")
    fn = pl.kernel(
        _band_body,
        out_type=jax.ShapeDtypeStruct((3, H, W), jnp.float32),
        mesh=mesh,
        scratch_types=[
            pltpu.VMEM((PCH,), jnp.float32),
            pltpu.VMEM((CAP + 16,), jnp.int32),
        ] + [pltpu.VMEM((CAP + 16,), jnp.float32)] * 8 + [
            pltpu.VMEM((FBA,), jnp.float32),
            pltpu.VMEM((FBA,), jnp.float32),
            pltpu.VMEM((FBA,), jnp.float32),
        ],
        compiler_params=pltpu.CompilerParams(
            use_tc_tiling_on_sc=False, needs_layout_passes=False),
    )
    return fn(px, py, a, b, c, fr, fg, fb)


def _pad_inputs(xyz, scl, rot, feat, opac):
    p = NP - N
    t = (jnp.arange(p, dtype=jnp.float32) + 0.5) / p
    tx = jnp.arctanh(t * 1.98 - 0.99)
    ty = jnp.arctanh(((t * 37.0) % 1.0) * 1.98 - 0.99)
    xyz_p = jnp.concatenate([xyz, jnp.stack([tx, ty], axis=-1)], axis=0)
    scl_p = jnp.concatenate([scl, jnp.zeros((p, 2), jnp.float32)], axis=0)
    rot_p = jnp.concatenate([rot, jnp.zeros((p, 1), jnp.float32)], axis=0)
    feat_p = jnp.concatenate([feat, jnp.zeros((p, 3), jnp.float32)], axis=0)
    opac_p = jnp.concatenate([opac, jnp.zeros((p, 1), jnp.float32)], axis=0)
    return xyz_p, scl_p, rot_p, feat_p, opac_p


def kernel(_xyz, _scaling, _rotation, _features_dc, _opacity):
    xyz, scl, rot, feat, opac = _pad_inputs(
        _xyz, _scaling, _rotation, _features_dc, _opacity)
    px, py, a, b, c, fr, fg, fb = _prep_call(xyz, scl, rot, feat, opac)
    img = _band_scatter(px, py, a, b, c, fr, fg, fb)
    return img.reshape(1, 3, H, W)
